# baseline (speedup 1.0000x reference)
import functools

import jax
import jax.numpy as jnp
from jax import lax
from jax.experimental import pallas as pl
from jax.experimental.pallas import tpu as pltpu
from jax.experimental.pallas import tpu_sc as plsc

H_HEADS = 12
GRID_SIZE = 3
SPLINE_ORDER = 3
NUM_EXPERTS = 8
TOP_K = 2
HID = 768
DFF = 768
SEQ = 2048
COEFF = GRID_SIZE + SPLINE_ORDER
NACT = COEFF + 1
KP = NACT * HID
DH = HID // H_HEADS
HALF = DH // 2

BM = 256
NBLK_M = SEQ // BM
NPAIR = TOP_K * SEQ
NPAD = NPAIR + NUM_EXPERTS * BM
NBLK_E = NPAD // BM

_H = 2.0 / GRID_SIZE


def _silu(x):
    return x / (1.0 + jnp.exp(-x))


def _dotT(a, w):
    return lax.dot_general(a, w, (((1,), (1,)), ((), ())),
                           preferred_element_type=jnp.float32)


def _dotT16(a, w):
    return lax.dot_general(a.astype(jnp.bfloat16), w.astype(jnp.bfloat16),
                           (((1,), (1,)), ((), ())),
                           preferred_element_type=jnp.float32)


def _spline_bases(x):
    g = [i * _H - 1.0 for i in range(-SPLINE_ORDER, GRID_SIZE + SPLINE_ORDER + 1)]
    b = [jnp.where((x >= g[j]) & (x < g[j + 1]), 1.0, 0.0).astype(jnp.float32)
         for j in range(len(g) - 1)]
    for p in range(1, SPLINE_ORDER + 1):
        denom = p * _H
        b = [(x - g[j]) / denom * b[j] + (g[j + p + 1] - x) / denom * b[j + 1]
             for j in range(len(b) - 1)]
    return b


def _acts(x):
    return [_silu(x)] + _spline_bases(x)


def _rmsnorm(x, w, eps=1e-6):
    return w * (x * lax.rsqrt(jnp.mean(x * x, axis=-1, keepdims=True) + eps))


def _qkv_body(x_ref, n1_ref, wb_ref, ws_ref, o_ref, acts_ref, acc_ref):
    c = pl.program_id(1)

    @pl.when(c == 0)
    def _():
        h = _rmsnorm(x_ref[...], n1_ref[...])
        a = _acts(h)
        for i in range(NACT):
            acts_ref[i] = a[i]
        acc_ref[...] = _dotT16(acts_ref[0], wb_ref[...])

    @pl.when(c > 0)
    def _():
        acc_ref[...] += _dotT16(acts_ref[c], ws_ref[0])

    @pl.when(c == NACT - 1)
    def _():
        o_ref[...] = acc_ref[...]


def _qkv_call(x2d, norm1_w, qkv_base, qspl_t):
    return pl.pallas_call(
        _qkv_body,
        grid=(NBLK_M, NACT),
        in_specs=[
            pl.BlockSpec((BM, HID), lambda m, c: (m, 0)),
            pl.BlockSpec((1, HID), lambda m, c: (0, 0)),
            pl.BlockSpec((3 * HID, HID), lambda m, c: (0, 0)),
            pl.BlockSpec((1, 3 * HID, HID),
                         lambda m, c: (jnp.maximum(c - 1, 0), 0, 0)),
        ],
        out_specs=pl.BlockSpec((BM, 3 * HID), lambda m, c: (m, 0)),
        out_shape=jax.ShapeDtypeStruct((SEQ, 3 * HID), jnp.float32),
        scratch_shapes=[
            pltpu.VMEM((NACT, BM, HID), jnp.float32),
            pltpu.VMEM((BM, 3 * HID), jnp.float32),
        ],
    )(x2d, norm1_w.reshape(1, HID), qkv_base, qspl_t)


def _attn_body(q_ref, k_ref, v_ref, cos_ref, sin_ref, o_ref):
    m = pl.program_id(1)
    q2 = q_ref[...]
    k2 = k_ref[...]
    v2 = v_ref[...]
    row = lax.broadcasted_iota(jnp.int32, (2 * DH, 2 * DH), 0)
    col = lax.broadcasted_iota(jnp.int32, (2 * DH, 2 * DH), 1)
    P = jnp.where(col == row - 2 * (row % 2) + 1, 1.0, 0.0).astype(jnp.float32)

    def rot(t, cs, sn):
        return t * cs + jnp.dot(t, P, preferred_element_type=jnp.float32) * sn

    qr = rot(q2, cos_ref[pl.ds(m * BM, BM), :], sin_ref[pl.ds(m * BM, BM), :])
    kr = rot(k2, cos_ref[...], sin_ref[...])
    ctxs = []
    for s in range(2):
        sl = slice(s * DH, (s + 1) * DH)
        scores = lax.dot_general(qr[:, sl], kr[:, sl], (((1,), (1,)), ((), ())),
                                 preferred_element_type=jnp.float32)
        scores = scores * (1.0 / (DH ** 0.5))
        mx = jnp.max(scores, axis=1, keepdims=True)
        p = jnp.exp(scores - mx)
        attn = p * (1.0 / jnp.sum(p, axis=1, keepdims=True))
        ctxs.append(jnp.dot(attn, v2[:, sl], preferred_element_type=jnp.float32))
    o_ref[...] = jnp.concatenate(ctxs, axis=1)


def _attn_call(qkv, cos_il2, sin_sg2):
    hp = H_HEADS // 2
    return pl.pallas_call(
        _attn_body,
        grid=(hp, NBLK_M),
        in_specs=[
            pl.BlockSpec((BM, 2 * DH), lambda h, m: (m, h)),
            pl.BlockSpec((SEQ, 2 * DH), lambda h, m: (0, hp + h)),
            pl.BlockSpec((SEQ, 2 * DH), lambda h, m: (0, 2 * hp + h)),
            pl.BlockSpec((SEQ, 2 * DH), lambda h, m: (0, 0)),
            pl.BlockSpec((SEQ, 2 * DH), lambda h, m: (0, 0)),
        ],
        out_specs=pl.BlockSpec((BM, 2 * DH), lambda h, m: (m, h)),
        out_shape=jax.ShapeDtypeStruct((SEQ, HID), jnp.float32),
    )(qkv, qkv, qkv, cos_il2, sin_sg2)


def _proj_gate_body(ctx_ref, x_ref, w_ref, b_ref, n2_ref, gw_ref,
                    x2_ref, h2_ref, sel_ref, wgt_ref):
    o = _dotT(ctx_ref[...], w_ref[...])
    x2 = x_ref[...] + o + b_ref[...]
    x2_ref[...] = x2
    h2 = _rmsnorm(x2, n2_ref[...])
    h2_ref[...] = h2
    logits = _dotT(h2, gw_ref[...])
    iota = lax.broadcasted_iota(jnp.int32, logits.shape, 1)
    m1 = jnp.max(logits, axis=1, keepdims=True)
    e1 = jnp.min(jnp.where(logits == m1, iota, NUM_EXPERTS),
                 axis=1, keepdims=True)
    masked = jnp.where(iota == e1, -jnp.inf, logits)
    m2 = jnp.max(masked, axis=1, keepdims=True)
    e2 = jnp.min(jnp.where(masked == m2, iota, NUM_EXPERTS),
                 axis=1, keepdims=True)
    t = jnp.exp(m2 - m1)
    w1 = 1.0 / (1.0 + t)
    w2 = 1.0 - w1
    sel_ref[...] = jnp.where(iota == 0, e1, jnp.where(iota == 1, e2, 0))
    wgt_ref[...] = jnp.where(iota == 0, w1, jnp.where(iota == 1, w2, 0.0))


def _proj_gate_call(ctx, x2d, wproj, out_b, norm2_w, gate_W):
    return pl.pallas_call(
        _proj_gate_body,
        grid=(NBLK_M,),
        in_specs=[
            pl.BlockSpec((BM, HID), lambda m: (m, 0)),
            pl.BlockSpec((BM, HID), lambda m: (m, 0)),
            pl.BlockSpec((HID, HID), lambda m: (0, 0)),
            pl.BlockSpec((1, HID), lambda m: (0, 0)),
            pl.BlockSpec((1, HID), lambda m: (0, 0)),
            pl.BlockSpec((NUM_EXPERTS, HID), lambda m: (0, 0)),
        ],
        out_specs=[
            pl.BlockSpec((BM, HID), lambda m: (m, 0)),
            pl.BlockSpec((BM, HID), lambda m: (m, 0)),
            pl.BlockSpec((BM, NUM_EXPERTS), lambda m: (m, 0)),
            pl.BlockSpec((BM, NUM_EXPERTS), lambda m: (m, 0)),
        ],
        out_shape=[
            jax.ShapeDtypeStruct((SEQ, HID), jnp.float32),
            jax.ShapeDtypeStruct((SEQ, HID), jnp.float32),
            jax.ShapeDtypeStruct((SEQ, NUM_EXPERTS), jnp.int32),
            jax.ShapeDtypeStruct((SEQ, NUM_EXPERTS), jnp.float32),
        ],
    )(ctx, x2d, wproj, out_b.reshape(1, HID), norm2_w.reshape(1, HID),
      gate_W)


def _sc_mesh():
    return plsc.VectorSubcoreMesh(core_axis_name="c", subcore_axis_name="s")


_GROWS = NPAD // 32
_GCHUNK = _GROWS // 2


def _sc_gather(h2, gidx):
    @functools.partial(
        pl.kernel, mesh=_sc_mesh(),
        out_type=jax.ShapeDtypeStruct((NPAD, HID), jnp.float32),
        scratch_types=[
            pltpu.VMEM((_GCHUNK,), jnp.int32),
            pltpu.VMEM((_GCHUNK, HID), jnp.float32),
            pltpu.SemaphoreType.DMA,
        ],
    )
    def k(h2_hbm, gidx_hbm, xg_hbm, idx_v, rows_v, sem):
        wid = lax.axis_index("s") * 2 + lax.axis_index("c")
        for j in range(2):
            base = wid * _GROWS + j * _GCHUNK
            pltpu.sync_copy(gidx_hbm.at[pl.ds(base, _GCHUNK)], idx_v)
            pltpu.async_copy(h2_hbm.at[idx_v], rows_v, sem).wait()
            pltpu.sync_copy(rows_v, xg_hbm.at[pl.ds(base, _GCHUNK)])

    return k(h2, gidx)


_CTOK = 32
_TPT = SEQ // 32
_NCC = _TPT // _CTOK


def _sc_combine(x2, ffw, p0, p1):
    @functools.partial(
        pl.kernel, mesh=_sc_mesh(),
        out_type=jax.ShapeDtypeStruct((SEQ, HID), jnp.float32),
        scratch_types=[
            pltpu.VMEM((_CTOK, HID), jnp.float32),
            pltpu.VMEM((_CTOK, HID), jnp.float32),
            pltpu.VMEM((_CTOK, HID), jnp.float32),
            pltpu.VMEM((_CTOK,), jnp.int32),
            pltpu.VMEM((_CTOK,), jnp.int32),
            pltpu.SemaphoreType.DMA,
        ],
    )
    def k(x2_hbm, ffw_hbm, p0_hbm, p1_hbm, out_hbm, acc, r0, r1, i0, i1, sem):
        wid = lax.axis_index("s") * 2 + lax.axis_index("c")
        for j in range(_NCC):
            base = wid * _TPT + j * _CTOK
            pltpu.sync_copy(p0_hbm.at[pl.ds(base, _CTOK)], i0)
            pltpu.sync_copy(p1_hbm.at[pl.ds(base, _CTOK)], i1)
            pltpu.sync_copy(x2_hbm.at[pl.ds(base, _CTOK)], acc)
            pltpu.async_copy(ffw_hbm.at[i0], r0, sem).wait()
            pltpu.async_copy(ffw_hbm.at[i1], r1, sem).wait()

            @pl.loop(0, _CTOK)
            def _(t):
                for c in range(HID // 16):
                    sl = pl.ds(c * 16, 16)
                    acc[t, sl] += r0[t, sl] + r1[t, sl]

            pltpu.sync_copy(acc, out_hbm.at[pl.ds(base, _CTOK)])

    return k(x2, ffw, p0, p1)


def _kan_acc(x, wb_ref, ws_ref):
    a = _acts(x)
    acc = _dotT16(a[0], wb_ref[0])
    for c in range(COEFF):
        acc += _dotT16(a[c + 1], ws_ref[0, c])
    return acc


def _e1a_body(eids_ref, xg_ref, wb_ref, ws_ref, o_ref):
    del eids_ref
    o_ref[...] = _kan_acc(xg_ref[...], wb_ref, ws_ref)


def _e1b_body(eids_ref, xg_ref, t1_ref, wb_ref, ws_ref, o_ref):
    del eids_ref
    o_ref[...] = _kan_acc(xg_ref[...], wb_ref, ws_ref) * t1_ref[...]


def _e2_body(eids_ref, u_ref, wbc_ref, wb_ref, ws_ref, o_ref):
    del eids_ref
    o_ref[...] = _kan_acc(u_ref[...], wb_ref, ws_ref) * wbc_ref[:, 0:1]


def _row_spec():
    return pl.BlockSpec((BM, HID), lambda b, eids: (b, 0))


def _wb_spec(out_d, in_d):
    return pl.BlockSpec((1, out_d, in_d), lambda b, eids: (eids[b], 0, 0))


def _ws_spec(out_d, in_d):
    return pl.BlockSpec((1, COEFF, out_d, in_d),
                        lambda b, eids: (eids[b], 0, 0, 0))


def _expert_call(body, ins, in_specs, out_dim, eids):
    grid_spec = pltpu.PrefetchScalarGridSpec(
        num_scalar_prefetch=1,
        grid=(NBLK_E,),
        in_specs=in_specs,
        out_specs=pl.BlockSpec((BM, out_dim), lambda b, eids: (b, 0)),
    )
    return pl.pallas_call(
        body,
        grid_spec=grid_spec,
        out_shape=jax.ShapeDtypeStruct((NPAD, out_dim), jnp.float32),
    )(eids, *ins)


def kernel(x, norm1_w, norm2_w, qkv_base, qkv_spline, out_W, out_b, gate_W,
           e_l1_base, e_l1_spline, e_l2_base, e_l2_spline, e_l3_base,
           e_l3_spline):
    x2d = x.reshape(SEQ, HID)

    r = jnp.arange(3 * HID)
    rowperm = (r % 768 // DH) * (3 * DH) + (r // 768) * DH + r % DH
    qkv_base_p = qkv_base[rowperm]
    qspl_t = qkv_spline.transpose(2, 0, 1)[:, rowperm, :]
    spl1_t = e_l1_spline.transpose(0, 3, 1, 2)
    spl2_t = e_l2_spline.transpose(0, 3, 1, 2)
    spl3_t = e_l3_spline.transpose(0, 3, 1, 2)
    freqs = 1.0 / (10000.0 ** (jnp.arange(HALF, dtype=jnp.float32) / HALF))
    angles = jnp.arange(SEQ, dtype=jnp.float32)[:, None] * freqs[None, :]
    cos_il2 = jnp.tile(jnp.repeat(jnp.cos(angles), 2, axis=1), (1, 2))
    sn = jnp.sin(angles)
    sin_sg2 = jnp.tile(jnp.stack([-sn, sn], axis=2).reshape(SEQ, DH), (1, 2))

    qkv = _qkv_call(x2d, norm1_w, qkv_base_p, qspl_t)
    ctx = _attn_call(qkv, cos_il2, sin_sg2)

    x2, h2, sel, wgt = _proj_gate_call(ctx, x2d, out_W, out_b, norm2_w,
                                       gate_W)

    e_flat = sel[:, :TOP_K].reshape(NPAIR)
    w_flat = wgt[:, :TOP_K].reshape(NPAIR)
    perm = jnp.argsort(e_flat, stable=True)
    sorted_e = e_flat[perm]
    counts = jnp.bincount(e_flat, length=NUM_EXPERTS)
    padded = ((counts + BM - 1) // BM) * BM
    shift = (jnp.cumsum(padded) - padded) - (jnp.cumsum(counts) - counts)
    dst = jnp.arange(NPAIR, dtype=jnp.int32) + shift[sorted_e].astype(jnp.int32)
    gidx = jnp.zeros(NPAD, jnp.int32).at[dst].set(
        (perm // TOP_K).astype(jnp.int32))
    w_pad = jnp.zeros(NPAD, jnp.float32).at[dst].set(w_flat[perm])
    blk_ends = jnp.cumsum(padded) // BM
    eids = jnp.minimum(
        jnp.sum(jnp.arange(NBLK_E)[:, None] >= blk_ends[None, :], axis=1),
        NUM_EXPERTS - 1).astype(jnp.int32)
    w_bcast = jnp.broadcast_to(w_pad[:, None], (NPAD, 128))
    pos = jnp.zeros(NPAIR, jnp.int32).at[perm].set(dst)
    p0 = pos[0::2]
    p1 = pos[1::2]

    xg = _sc_gather(h2, gidx)
    t1 = _expert_call(_e1a_body, (xg, e_l1_base, spl1_t),
                      [_row_spec(), _wb_spec(DFF, HID), _ws_spec(DFF, HID)],
                      DFF, eids)
    u = _expert_call(_e1b_body, (xg, t1, e_l2_base, spl2_t),
                     [_row_spec(),
                      pl.BlockSpec((BM, DFF), lambda b, eids: (b, 0)),
                      _wb_spec(DFF, HID), _ws_spec(DFF, HID)], DFF, eids)
    ffw = _expert_call(_e2_body, (u, w_bcast, e_l3_base, spl3_t),
                       [pl.BlockSpec((BM, DFF), lambda b, eids: (b, 0)),
                        pl.BlockSpec((BM, 128), lambda b, eids: (b, 0)),
                        _wb_spec(HID, DFF), _ws_spec(HID, DFF)],
                       HID, eids)

    out2d = _sc_combine(x2, ffw, p0, p1)
    return out2d.reshape(1, SEQ, HID)


# speedup vs baseline: 1.1329x; 1.1329x over previous
import functools

import jax
import jax.numpy as jnp
from jax import lax
from jax.experimental import pallas as pl
from jax.experimental.pallas import tpu as pltpu
from jax.experimental.pallas import tpu_sc as plsc

H_HEADS = 12
GRID_SIZE = 3
SPLINE_ORDER = 3
NUM_EXPERTS = 8
TOP_K = 2
HID = 768
DFF = 768
SEQ = 2048
COEFF = GRID_SIZE + SPLINE_ORDER
NACT = COEFF + 1
KP = NACT * HID
DH = HID // H_HEADS
HALF = DH // 2

BM = 256
NBLK_M = SEQ // BM
BMQ = 512
NBLK_Q = SEQ // BMQ
NPAIR = TOP_K * SEQ
NPAD = NPAIR + NUM_EXPERTS * BM
NBLK_E = NPAD // BM

_H = 2.0 / GRID_SIZE


def _silu(x):
    return x / (1.0 + jnp.exp(-x))


def _dotT(a, w):
    return lax.dot_general(a, w, (((1,), (1,)), ((), ())),
                           preferred_element_type=jnp.float32)


def _dotT16(a, w):
    return lax.dot_general(a.astype(jnp.bfloat16), w.astype(jnp.bfloat16),
                           (((1,), (1,)), ((), ())),
                           preferred_element_type=jnp.float32)


def _spline_bases(x):
    g = [i * _H - 1.0 for i in range(-SPLINE_ORDER, GRID_SIZE + SPLINE_ORDER + 1)]
    b = [jnp.where((x >= g[j]) & (x < g[j + 1]), 1.0, 0.0)
         .astype(jnp.bfloat16) for j in range(len(g) - 1)]
    x = x.astype(jnp.bfloat16)
    for p in range(1, SPLINE_ORDER + 1):
        denom = p * _H
        b = [(x - g[j]) / denom * b[j] + (g[j + p + 1] - x) / denom * b[j + 1]
             for j in range(len(b) - 1)]
    return b


def _acts(x):
    return [_silu(x.astype(jnp.bfloat16))] + _spline_bases(x)


def _rmsnorm(x, w, eps=1e-6):
    return w * (x * lax.rsqrt(jnp.mean(x * x, axis=-1, keepdims=True) + eps))


def _qkv_body(x_ref, n1_ref, wb_ref, ws_ref, o_ref, acts_ref, acc_ref):
    c = pl.program_id(1)

    @pl.when(c == 0)
    def _():
        h = _rmsnorm(x_ref[...], n1_ref[...])
        a = _acts(h)
        for i in range(NACT):
            acts_ref[i] = a[i]
        acc_ref[...] = _dotT16(acts_ref[0], wb_ref[...])

    @pl.when(c > 0)
    def _():
        acc_ref[...] += _dotT16(acts_ref[c], ws_ref[0])

    @pl.when(c == NACT - 1)
    def _():
        o_ref[...] = acc_ref[...]


def _qkv_call(x2d, norm1_w, qkv_base, qspl_t):
    return pl.pallas_call(
        _qkv_body,
        grid=(NBLK_Q, NACT),
        in_specs=[
            pl.BlockSpec((BMQ, HID), lambda m, c: (m, 0)),
            pl.BlockSpec((1, HID), lambda m, c: (0, 0)),
            pl.BlockSpec((3 * HID, HID), lambda m, c: (0, 0)),
            pl.BlockSpec((1, 3 * HID, HID),
                         lambda m, c: (jnp.maximum(c - 1, 0), 0, 0)),
        ],
        out_specs=pl.BlockSpec((BMQ, 3 * HID), lambda m, c: (m, 0)),
        out_shape=jax.ShapeDtypeStruct((SEQ, 3 * HID), jnp.float32),
        scratch_shapes=[
            pltpu.VMEM((NACT, BMQ, HID), jnp.bfloat16),
            pltpu.VMEM((BMQ, 3 * HID), jnp.float32),
        ],
    )(x2d, norm1_w.reshape(1, HID), qkv_base, qspl_t)


def _attn_body(q_ref, k_ref, v_ref, cos_ref, sin_ref, o_ref):
    m = pl.program_id(1)
    q2 = q_ref[...]
    k2 = k_ref[...]
    v2 = v_ref[...]
    row = lax.broadcasted_iota(jnp.int32, (2 * DH, 2 * DH), 0)
    col = lax.broadcasted_iota(jnp.int32, (2 * DH, 2 * DH), 1)
    P = jnp.where(col == row - 2 * (row % 2) + 1, 1.0, 0.0).astype(jnp.float32)

    def rot(t, cs, sn):
        return t * cs + jnp.dot(t, P, preferred_element_type=jnp.float32) * sn

    qr = rot(q2, cos_ref[pl.ds(m * BM, BM), :], sin_ref[pl.ds(m * BM, BM), :])
    kr = rot(k2, cos_ref[...], sin_ref[...])
    ctxs = []
    for s in range(2):
        sl = slice(s * DH, (s + 1) * DH)
        scores = lax.dot_general(qr[:, sl], kr[:, sl], (((1,), (1,)), ((), ())),
                                 preferred_element_type=jnp.float32)
        scores = scores * (1.0 / (DH ** 0.5))
        mx = jnp.max(scores, axis=1, keepdims=True)
        p = jnp.exp(scores - mx)
        attn = p * (1.0 / jnp.sum(p, axis=1, keepdims=True))
        ctxs.append(jnp.dot(attn, v2[:, sl], preferred_element_type=jnp.float32))
    o_ref[...] = jnp.concatenate(ctxs, axis=1)


def _attn_call(qkv, cos_il2, sin_sg2):
    hp = H_HEADS // 2
    return pl.pallas_call(
        _attn_body,
        grid=(hp, NBLK_M),
        in_specs=[
            pl.BlockSpec((BM, 2 * DH), lambda h, m: (m, h)),
            pl.BlockSpec((SEQ, 2 * DH), lambda h, m: (0, hp + h)),
            pl.BlockSpec((SEQ, 2 * DH), lambda h, m: (0, 2 * hp + h)),
            pl.BlockSpec((SEQ, 2 * DH), lambda h, m: (0, 0)),
            pl.BlockSpec((SEQ, 2 * DH), lambda h, m: (0, 0)),
        ],
        out_specs=pl.BlockSpec((BM, 2 * DH), lambda h, m: (m, h)),
        out_shape=jax.ShapeDtypeStruct((SEQ, HID), jnp.float32),
    )(qkv, qkv, qkv, cos_il2, sin_sg2)


def _proj_gate_body(ctx_ref, x_ref, w_ref, b_ref, n2_ref, gw_ref,
                    x2_ref, h2_ref, sel_ref, wgt_ref):
    o = _dotT(ctx_ref[...], w_ref[...])
    x2 = x_ref[...] + o + b_ref[...]
    x2_ref[...] = x2
    h2 = _rmsnorm(x2, n2_ref[...])
    h2_ref[...] = h2
    logits = _dotT(h2, gw_ref[...])
    iota = lax.broadcasted_iota(jnp.int32, logits.shape, 1)
    m1 = jnp.max(logits, axis=1, keepdims=True)
    e1 = jnp.min(jnp.where(logits == m1, iota, NUM_EXPERTS),
                 axis=1, keepdims=True)
    masked = jnp.where(iota == e1, -jnp.inf, logits)
    m2 = jnp.max(masked, axis=1, keepdims=True)
    e2 = jnp.min(jnp.where(masked == m2, iota, NUM_EXPERTS),
                 axis=1, keepdims=True)
    t = jnp.exp(m2 - m1)
    w1 = 1.0 / (1.0 + t)
    w2 = 1.0 - w1
    sel_ref[...] = jnp.where(iota == 0, e1, jnp.where(iota == 1, e2, 0))
    wgt_ref[...] = jnp.where(iota == 0, w1, jnp.where(iota == 1, w2, 0.0))


def _proj_gate_call(ctx, x2d, wproj, out_b, norm2_w, gate_W):
    return pl.pallas_call(
        _proj_gate_body,
        grid=(NBLK_M,),
        in_specs=[
            pl.BlockSpec((BM, HID), lambda m: (m, 0)),
            pl.BlockSpec((BM, HID), lambda m: (m, 0)),
            pl.BlockSpec((HID, HID), lambda m: (0, 0)),
            pl.BlockSpec((1, HID), lambda m: (0, 0)),
            pl.BlockSpec((1, HID), lambda m: (0, 0)),
            pl.BlockSpec((NUM_EXPERTS, HID), lambda m: (0, 0)),
        ],
        out_specs=[
            pl.BlockSpec((BM, HID), lambda m: (m, 0)),
            pl.BlockSpec((BM, HID), lambda m: (m, 0)),
            pl.BlockSpec((BM, NUM_EXPERTS), lambda m: (m, 0)),
            pl.BlockSpec((BM, NUM_EXPERTS), lambda m: (m, 0)),
        ],
        out_shape=[
            jax.ShapeDtypeStruct((SEQ, HID), jnp.float32),
            jax.ShapeDtypeStruct((SEQ, HID), jnp.float32),
            jax.ShapeDtypeStruct((SEQ, NUM_EXPERTS), jnp.int32),
            jax.ShapeDtypeStruct((SEQ, NUM_EXPERTS), jnp.float32),
        ],
    )(ctx, x2d, wproj, out_b.reshape(1, HID), norm2_w.reshape(1, HID),
      gate_W)


def _sc_mesh():
    return plsc.VectorSubcoreMesh(core_axis_name="c", subcore_axis_name="s")


_GROWS = NPAD // 32
_GCHUNK = _GROWS // 2


def _sc_gather(h2, gidx):
    @functools.partial(
        pl.kernel, mesh=_sc_mesh(),
        out_type=jax.ShapeDtypeStruct((NPAD, HID), jnp.float32),
        scratch_types=[
            pltpu.VMEM((_GCHUNK,), jnp.int32),
            pltpu.VMEM((_GCHUNK, HID), jnp.float32),
            pltpu.SemaphoreType.DMA,
        ],
    )
    def k(h2_hbm, gidx_hbm, xg_hbm, idx_v, rows_v, sem):
        wid = lax.axis_index("s") * 2 + lax.axis_index("c")
        for j in range(2):
            base = wid * _GROWS + j * _GCHUNK
            pltpu.sync_copy(gidx_hbm.at[pl.ds(base, _GCHUNK)], idx_v)
            pltpu.async_copy(h2_hbm.at[idx_v], rows_v, sem).wait()
            pltpu.sync_copy(rows_v, xg_hbm.at[pl.ds(base, _GCHUNK)])

    return k(h2, gidx)


_CTOK = 32
_TPT = SEQ // 32
_NCC = _TPT // _CTOK


def _sc_combine(x2, ffw, p0, p1):
    @functools.partial(
        pl.kernel, mesh=_sc_mesh(),
        out_type=jax.ShapeDtypeStruct((SEQ, HID), jnp.float32),
        scratch_types=[
            pltpu.VMEM((_CTOK, HID), jnp.float32),
            pltpu.VMEM((_CTOK, HID), jnp.float32),
            pltpu.VMEM((_CTOK, HID), jnp.float32),
            pltpu.VMEM((_CTOK,), jnp.int32),
            pltpu.VMEM((_CTOK,), jnp.int32),
            pltpu.SemaphoreType.DMA,
        ],
    )
    def k(x2_hbm, ffw_hbm, p0_hbm, p1_hbm, out_hbm, acc, r0, r1, i0, i1, sem):
        wid = lax.axis_index("s") * 2 + lax.axis_index("c")
        for j in range(_NCC):
            base = wid * _TPT + j * _CTOK
            pltpu.sync_copy(p0_hbm.at[pl.ds(base, _CTOK)], i0)
            pltpu.sync_copy(p1_hbm.at[pl.ds(base, _CTOK)], i1)
            pltpu.sync_copy(x2_hbm.at[pl.ds(base, _CTOK)], acc)
            pltpu.async_copy(ffw_hbm.at[i0], r0, sem).wait()
            pltpu.async_copy(ffw_hbm.at[i1], r1, sem).wait()

            @pl.loop(0, _CTOK)
            def _(t):
                for c in range(HID // 16):
                    sl = pl.ds(c * 16, 16)
                    acc[t, sl] += r0[t, sl] + r1[t, sl]

            pltpu.sync_copy(acc, out_hbm.at[pl.ds(base, _CTOK)])

    return k(x2, ffw, p0, p1)


def _kan_acc(x, wb_ref, ws_ref):
    a = _acts(x)
    acc = _dotT16(a[0], wb_ref[0])
    for c in range(COEFF):
        acc += _dotT16(a[c + 1], ws_ref[0, c])
    return acc


def _e1a_body(eids_ref, xg_ref, wb_ref, ws_ref, o_ref):
    del eids_ref
    o_ref[...] = _kan_acc(xg_ref[...], wb_ref, ws_ref)


def _e1b_body(eids_ref, xg_ref, t1_ref, wb_ref, ws_ref, o_ref):
    del eids_ref
    o_ref[...] = _kan_acc(xg_ref[...], wb_ref, ws_ref) * t1_ref[...]


def _e2_body(eids_ref, u_ref, wbc_ref, wb_ref, ws_ref, o_ref):
    del eids_ref
    o_ref[...] = _kan_acc(u_ref[...], wb_ref, ws_ref) * wbc_ref[:, 0:1]


def _row_spec():
    return pl.BlockSpec((BM, HID), lambda b, eids: (b, 0))


def _wb_spec(out_d, in_d):
    return pl.BlockSpec((1, out_d, in_d), lambda b, eids: (eids[b], 0, 0))


def _ws_spec(out_d, in_d):
    return pl.BlockSpec((1, COEFF, out_d, in_d),
                        lambda b, eids: (eids[b], 0, 0, 0))


def _expert_call(body, ins, in_specs, out_dim, eids):
    grid_spec = pltpu.PrefetchScalarGridSpec(
        num_scalar_prefetch=1,
        grid=(NBLK_E,),
        in_specs=in_specs,
        out_specs=pl.BlockSpec((BM, out_dim), lambda b, eids: (b, 0)),
    )
    return pl.pallas_call(
        body,
        grid_spec=grid_spec,
        out_shape=jax.ShapeDtypeStruct((NPAD, out_dim), jnp.float32),
    )(eids, *ins)


def kernel(x, norm1_w, norm2_w, qkv_base, qkv_spline, out_W, out_b, gate_W,
           e_l1_base, e_l1_spline, e_l2_base, e_l2_spline, e_l3_base,
           e_l3_spline):
    x2d = x.reshape(SEQ, HID)

    r = jnp.arange(3 * HID)
    rowperm = (r % 768 // DH) * (3 * DH) + (r // 768) * DH + r % DH
    qkv_base_p = qkv_base[rowperm]
    qspl_t = qkv_spline.transpose(2, 0, 1)[:, rowperm, :]
    spl1_t = e_l1_spline.transpose(0, 3, 1, 2)
    spl2_t = e_l2_spline.transpose(0, 3, 1, 2)
    spl3_t = e_l3_spline.transpose(0, 3, 1, 2)
    freqs = 1.0 / (10000.0 ** (jnp.arange(HALF, dtype=jnp.float32) / HALF))
    angles = jnp.arange(SEQ, dtype=jnp.float32)[:, None] * freqs[None, :]
    cos_il2 = jnp.tile(jnp.repeat(jnp.cos(angles), 2, axis=1), (1, 2))
    sn = jnp.sin(angles)
    sin_sg2 = jnp.tile(jnp.stack([-sn, sn], axis=2).reshape(SEQ, DH), (1, 2))

    qkv = _qkv_call(x2d, norm1_w, qkv_base_p, qspl_t)
    ctx = _attn_call(qkv, cos_il2, sin_sg2)

    x2, h2, sel, wgt = _proj_gate_call(ctx, x2d, out_W, out_b, norm2_w,
                                       gate_W)

    e_flat = sel[:, :TOP_K].reshape(NPAIR)
    w_flat = wgt[:, :TOP_K].reshape(NPAIR)
    perm = jnp.argsort(e_flat, stable=True)
    sorted_e = e_flat[perm]
    counts = jnp.bincount(e_flat, length=NUM_EXPERTS)
    padded = ((counts + BM - 1) // BM) * BM
    shift = (jnp.cumsum(padded) - padded) - (jnp.cumsum(counts) - counts)
    dst = jnp.arange(NPAIR, dtype=jnp.int32) + shift[sorted_e].astype(jnp.int32)
    gidx = jnp.zeros(NPAD, jnp.int32).at[dst].set(
        (perm // TOP_K).astype(jnp.int32))
    w_pad = jnp.zeros(NPAD, jnp.float32).at[dst].set(w_flat[perm])
    blk_ends = jnp.cumsum(padded) // BM
    eids = jnp.minimum(
        jnp.sum(jnp.arange(NBLK_E)[:, None] >= blk_ends[None, :], axis=1),
        NUM_EXPERTS - 1).astype(jnp.int32)
    w_bcast = jnp.broadcast_to(w_pad[:, None], (NPAD, 128))
    pos = jnp.zeros(NPAIR, jnp.int32).at[perm].set(dst)
    p0 = pos[0::2]
    p1 = pos[1::2]

    xg = _sc_gather(h2, gidx)
    t1 = _expert_call(_e1a_body, (xg, e_l1_base, spl1_t),
                      [_row_spec(), _wb_spec(DFF, HID), _ws_spec(DFF, HID)],
                      DFF, eids)
    u = _expert_call(_e1b_body, (xg, t1, e_l2_base, spl2_t),
                     [_row_spec(),
                      pl.BlockSpec((BM, DFF), lambda b, eids: (b, 0)),
                      _wb_spec(DFF, HID), _ws_spec(DFF, HID)], DFF, eids)
    ffw = _expert_call(_e2_body, (u, w_bcast, e_l3_base, spl3_t),
                       [pl.BlockSpec((BM, DFF), lambda b, eids: (b, 0)),
                        pl.BlockSpec((BM, 128), lambda b, eids: (b, 0)),
                        _wb_spec(HID, DFF), _ws_spec(HID, DFF)],
                       HID, eids)

    out2d = _sc_combine(x2, ffw, p0, p1)
    return out2d.reshape(1, SEQ, HID)


# speedup vs baseline: 1.3210x; 1.1660x over previous
import functools

import jax
import jax.numpy as jnp
from jax import lax
from jax.experimental import pallas as pl
from jax.experimental.pallas import tpu as pltpu
from jax.experimental.pallas import tpu_sc as plsc

H_HEADS = 12
GRID_SIZE = 3
SPLINE_ORDER = 3
NUM_EXPERTS = 8
TOP_K = 2
HID = 768
DFF = 768
SEQ = 2048
COEFF = GRID_SIZE + SPLINE_ORDER
NACT = COEFF + 1
KP = NACT * HID
DH = HID // H_HEADS
HALF = DH // 2

BM = 256
NBLK_M = SEQ // BM
BMQ = 512
NBLK_Q = SEQ // BMQ
NPAIR = TOP_K * SEQ
NPAD = NPAIR + NUM_EXPERTS * BM
NBLK_E = NPAD // BM

_H = 2.0 / GRID_SIZE


def _silu(x):
    return x / (1.0 + jnp.exp(-x))


def _dotT(a, w):
    return lax.dot_general(a, w, (((1,), (1,)), ((), ())),
                           preferred_element_type=jnp.float32)


def _dotT16(a, w):
    return lax.dot_general(a.astype(jnp.bfloat16), w.astype(jnp.bfloat16),
                           (((1,), (1,)), ((), ())),
                           preferred_element_type=jnp.float32)


def _spline_bases(x):
    g = [i * _H - 1.0 for i in range(-SPLINE_ORDER, GRID_SIZE + SPLINE_ORDER + 1)]
    b = [jnp.where((x >= g[j]) & (x < g[j + 1]), 1.0, 0.0)
         .astype(jnp.bfloat16) for j in range(len(g) - 1)]
    x = x.astype(jnp.bfloat16)
    for p in range(1, SPLINE_ORDER + 1):
        denom = p * _H
        b = [(x - g[j]) / denom * b[j] + (g[j + p + 1] - x) / denom * b[j + 1]
             for j in range(len(b) - 1)]
    return b


def _acts(x):
    return [_silu(x.astype(jnp.bfloat16))] + _spline_bases(x)


def _rmsnorm(x, w, eps=1e-6):
    return w * (x * lax.rsqrt(jnp.mean(x * x, axis=-1, keepdims=True) + eps))


def _qkv_body(x_ref, n1_ref, wb_ref, ws_ref, o_ref, acts_ref, acc_ref):
    c = pl.program_id(1)

    @pl.when(c == 0)
    def _():
        h = _rmsnorm(x_ref[...], n1_ref[...])
        a = _acts(h)
        for i in range(NACT):
            acts_ref[i] = a[i]
        acc_ref[...] = _dotT16(acts_ref[0], wb_ref[...])

    @pl.when(c > 0)
    def _():
        acc_ref[...] += _dotT16(acts_ref[c], ws_ref[0])

    @pl.when(c == NACT - 1)
    def _():
        o_ref[...] = acc_ref[...]


def _qkv_call(x2d, norm1_w, qkv_base, qspl_t):
    return pl.pallas_call(
        _qkv_body,
        grid=(NBLK_Q, NACT),
        in_specs=[
            pl.BlockSpec((BMQ, HID), lambda m, c: (m, 0)),
            pl.BlockSpec((1, HID), lambda m, c: (0, 0)),
            pl.BlockSpec((3 * HID, HID), lambda m, c: (0, 0)),
            pl.BlockSpec((1, 3 * HID, HID),
                         lambda m, c: (jnp.maximum(c - 1, 0), 0, 0)),
        ],
        out_specs=pl.BlockSpec((BMQ, 3 * HID), lambda m, c: (m, 0)),
        out_shape=jax.ShapeDtypeStruct((SEQ, 3 * HID), jnp.float32),
        scratch_shapes=[
            pltpu.VMEM((NACT, BMQ, HID), jnp.bfloat16),
            pltpu.VMEM((BMQ, 3 * HID), jnp.float32),
        ],
    )(x2d, norm1_w.reshape(1, HID), qkv_base, qspl_t)


def _attn_body(q_ref, k_ref, v_ref, cos_ref, sin_ref, o_ref):
    m = pl.program_id(1)
    q2 = q_ref[...]
    k2 = k_ref[...]
    v2 = v_ref[...]
    row = lax.broadcasted_iota(jnp.int32, (2 * DH, 2 * DH), 0)
    col = lax.broadcasted_iota(jnp.int32, (2 * DH, 2 * DH), 1)
    P = jnp.where(col == row - 2 * (row % 2) + 1, 1.0, 0.0).astype(jnp.float32)

    def rot(t, cs, sn):
        return t * cs + jnp.dot(t, P, preferred_element_type=jnp.float32) * sn

    qr = rot(q2, cos_ref[pl.ds(m * BM, BM), :], sin_ref[pl.ds(m * BM, BM), :])
    kr = rot(k2, cos_ref[...], sin_ref[...])
    ctxs = []
    for s in range(2):
        sl = slice(s * DH, (s + 1) * DH)
        scores = lax.dot_general(qr[:, sl], kr[:, sl], (((1,), (1,)), ((), ())),
                                 preferred_element_type=jnp.float32)
        scores = scores * (1.0 / (DH ** 0.5))
        mx = jnp.max(scores, axis=1, keepdims=True)
        p = jnp.exp(scores - mx)
        attn = p * (1.0 / jnp.sum(p, axis=1, keepdims=True))
        ctxs.append(jnp.dot(attn, v2[:, sl], preferred_element_type=jnp.float32))
    o_ref[...] = jnp.concatenate(ctxs, axis=1)


def _attn_call(qkv, cos_il2, sin_sg2):
    hp = H_HEADS // 2
    return pl.pallas_call(
        _attn_body,
        grid=(hp, NBLK_M),
        in_specs=[
            pl.BlockSpec((BM, 2 * DH), lambda h, m: (m, h)),
            pl.BlockSpec((SEQ, 2 * DH), lambda h, m: (0, hp + h)),
            pl.BlockSpec((SEQ, 2 * DH), lambda h, m: (0, 2 * hp + h)),
            pl.BlockSpec((SEQ, 2 * DH), lambda h, m: (0, 0)),
            pl.BlockSpec((SEQ, 2 * DH), lambda h, m: (0, 0)),
        ],
        out_specs=pl.BlockSpec((BM, 2 * DH), lambda h, m: (m, h)),
        out_shape=jax.ShapeDtypeStruct((SEQ, HID), jnp.float32),
    )(qkv, qkv, qkv, cos_il2, sin_sg2)


def _proj_gate_body(ctx_ref, x_ref, w_ref, b_ref, n2_ref, gw_ref,
                    x2_ref, h2_ref, sel_ref, wgt_ref):
    o = _dotT(ctx_ref[...], w_ref[...])
    x2 = x_ref[...] + o + b_ref[...]
    x2_ref[...] = x2
    h2 = _rmsnorm(x2, n2_ref[...])
    h2_ref[...] = h2
    logits = _dotT(h2, gw_ref[...])
    iota = lax.broadcasted_iota(jnp.int32, logits.shape, 1)
    m1 = jnp.max(logits, axis=1, keepdims=True)
    e1 = jnp.min(jnp.where(logits == m1, iota, NUM_EXPERTS),
                 axis=1, keepdims=True)
    masked = jnp.where(iota == e1, -jnp.inf, logits)
    m2 = jnp.max(masked, axis=1, keepdims=True)
    e2 = jnp.min(jnp.where(masked == m2, iota, NUM_EXPERTS),
                 axis=1, keepdims=True)
    t = jnp.exp(m2 - m1)
    w1 = 1.0 / (1.0 + t)
    w2 = 1.0 - w1
    sel_ref[...] = jnp.where(iota == 0, e1, jnp.where(iota == 1, e2, 0))
    wgt_ref[...] = jnp.where(iota == 0, w1, jnp.where(iota == 1, w2, 0.0))


def _proj_gate_call(ctx, x2d, wproj, out_b, norm2_w, gate_W):
    return pl.pallas_call(
        _proj_gate_body,
        grid=(NBLK_M,),
        in_specs=[
            pl.BlockSpec((BM, HID), lambda m: (m, 0)),
            pl.BlockSpec((BM, HID), lambda m: (m, 0)),
            pl.BlockSpec((HID, HID), lambda m: (0, 0)),
            pl.BlockSpec((1, HID), lambda m: (0, 0)),
            pl.BlockSpec((1, HID), lambda m: (0, 0)),
            pl.BlockSpec((NUM_EXPERTS, HID), lambda m: (0, 0)),
        ],
        out_specs=[
            pl.BlockSpec((BM, HID), lambda m: (m, 0)),
            pl.BlockSpec((BM, HID), lambda m: (m, 0)),
            pl.BlockSpec((BM, NUM_EXPERTS), lambda m: (m, 0)),
            pl.BlockSpec((BM, NUM_EXPERTS), lambda m: (m, 0)),
        ],
        out_shape=[
            jax.ShapeDtypeStruct((SEQ, HID), jnp.float32),
            jax.ShapeDtypeStruct((SEQ, HID), jnp.float32),
            jax.ShapeDtypeStruct((SEQ, NUM_EXPERTS), jnp.int32),
            jax.ShapeDtypeStruct((SEQ, NUM_EXPERTS), jnp.float32),
        ],
    )(ctx, x2d, wproj, out_b.reshape(1, HID), norm2_w.reshape(1, HID),
      gate_W)


def _sc_mesh():
    return plsc.VectorSubcoreMesh(core_axis_name="c", subcore_axis_name="s")


_DTOK = SEQ // 32


def _sc_dispatch(h2, p0, p1, w1b, w2b):
    @functools.partial(
        pl.kernel, mesh=_sc_mesh(),
        out_type=(jax.ShapeDtypeStruct((NPAD, HID), jnp.float32),
                  jax.ShapeDtypeStruct((NPAD, 128), jnp.float32)),
        scratch_types=[
            pltpu.VMEM((_DTOK, HID), jnp.float32),
            pltpu.VMEM((_DTOK, 128), jnp.float32),
            pltpu.VMEM((1, _DTOK), jnp.int32),
            pltpu.VMEM((1, _DTOK), jnp.int32),
        ],
    )
    def k(h2_hbm, p0_hbm, p1_hbm, w1_hbm, w2_hbm, xg_hbm, wp_hbm,
          rows_v, wrow_v, i0, i1):
        wid = lax.axis_index("s") * 2 + lax.axis_index("c")
        pltpu.sync_copy(p0_hbm.at[pl.ds(wid, 1)], i0)
        pltpu.sync_copy(p1_hbm.at[pl.ds(wid, 1)], i1)
        pltpu.sync_copy(h2_hbm.at[pl.ds(wid * _DTOK, _DTOK)], rows_v)
        pltpu.sync_copy(rows_v, xg_hbm.at[i0.at[0]])
        pltpu.sync_copy(rows_v, xg_hbm.at[i1.at[0]])
        pltpu.sync_copy(w1_hbm.at[pl.ds(wid * _DTOK, _DTOK)], wrow_v)
        pltpu.sync_copy(wrow_v, wp_hbm.at[i0.at[0]])
        pltpu.sync_copy(w2_hbm.at[pl.ds(wid * _DTOK, _DTOK)], wrow_v)
        pltpu.sync_copy(wrow_v, wp_hbm.at[i1.at[0]])

    return k(h2, p0, p1, w1b, w2b)


_CTOK = 32
_TPT = SEQ // 32
_NCC = _TPT // _CTOK


def _sc_combine(x2, ffw, p0, p1):
    @functools.partial(
        pl.kernel, mesh=_sc_mesh(),
        out_type=jax.ShapeDtypeStruct((SEQ, HID), jnp.float32),
        scratch_types=[
            pltpu.VMEM((_CTOK, HID), jnp.float32),
            pltpu.VMEM((_CTOK, HID), jnp.float32),
            pltpu.VMEM((_CTOK, HID), jnp.float32),
            pltpu.VMEM((_CTOK,), jnp.int32),
            pltpu.VMEM((_CTOK,), jnp.int32),
            pltpu.SemaphoreType.DMA,
        ],
    )
    def k(x2_hbm, ffw_hbm, p0_hbm, p1_hbm, out_hbm, acc, r0, r1, i0, i1, sem):
        wid = lax.axis_index("s") * 2 + lax.axis_index("c")
        for j in range(_NCC):
            base = wid * _TPT + j * _CTOK
            pltpu.sync_copy(p0_hbm.at[pl.ds(base, _CTOK)], i0)
            pltpu.sync_copy(p1_hbm.at[pl.ds(base, _CTOK)], i1)
            pltpu.sync_copy(x2_hbm.at[pl.ds(base, _CTOK)], acc)
            pltpu.async_copy(ffw_hbm.at[i0], r0, sem).wait()
            pltpu.async_copy(ffw_hbm.at[i1], r1, sem).wait()

            @pl.loop(0, _CTOK)
            def _(t):
                for c in range(HID // 16):
                    sl = pl.ds(c * 16, 16)
                    acc[t, sl] += r0[t, sl] + r1[t, sl]

            pltpu.sync_copy(acc, out_hbm.at[pl.ds(base, _CTOK)])

    return k(x2, ffw, p0, p1)


def _kan_acc(x, wb_ref, ws_ref):
    a = _acts(x)
    acc = _dotT16(a[0], wb_ref[0])
    for c in range(COEFF):
        acc += _dotT16(a[c + 1], ws_ref[0, c])
    return acc


def _e1a_body(eids_ref, xg_ref, wb_ref, ws_ref, o_ref):
    del eids_ref
    o_ref[...] = _kan_acc(xg_ref[...], wb_ref, ws_ref)


def _e1b_body(eids_ref, xg_ref, t1_ref, wb_ref, ws_ref, o_ref):
    del eids_ref
    o_ref[...] = _kan_acc(xg_ref[...], wb_ref, ws_ref) * t1_ref[...]


def _e2_body(eids_ref, u_ref, wbc_ref, wb_ref, ws_ref, o_ref):
    del eids_ref
    o_ref[...] = _kan_acc(u_ref[...], wb_ref, ws_ref) * wbc_ref[:, 0:1]


def _row_spec():
    return pl.BlockSpec((BM, HID), lambda b, eids: (b, 0))


def _wb_spec(out_d, in_d):
    return pl.BlockSpec((1, out_d, in_d), lambda b, eids: (eids[b], 0, 0))


def _ws_spec(out_d, in_d):
    return pl.BlockSpec((1, COEFF, out_d, in_d),
                        lambda b, eids: (eids[b], 0, 0, 0))


def _expert_call(body, ins, in_specs, out_dim, eids):
    grid_spec = pltpu.PrefetchScalarGridSpec(
        num_scalar_prefetch=1,
        grid=(NBLK_E,),
        in_specs=in_specs,
        out_specs=pl.BlockSpec((BM, out_dim), lambda b, eids: (b, 0)),
    )
    return pl.pallas_call(
        body,
        grid_spec=grid_spec,
        out_shape=jax.ShapeDtypeStruct((NPAD, out_dim), jnp.float32),
    )(eids, *ins)


def kernel(x, norm1_w, norm2_w, qkv_base, qkv_spline, out_W, out_b, gate_W,
           e_l1_base, e_l1_spline, e_l2_base, e_l2_spline, e_l3_base,
           e_l3_spline):
    x2d = x.reshape(SEQ, HID)

    r = jnp.arange(3 * HID)
    rowperm = (r % 768 // DH) * (3 * DH) + (r // 768) * DH + r % DH
    qkv_base_p = qkv_base[rowperm]
    qspl_t = qkv_spline.transpose(2, 0, 1)[:, rowperm, :]
    spl1_t = e_l1_spline.transpose(0, 3, 1, 2)
    spl2_t = e_l2_spline.transpose(0, 3, 1, 2)
    spl3_t = e_l3_spline.transpose(0, 3, 1, 2)
    freqs = 1.0 / (10000.0 ** (jnp.arange(HALF, dtype=jnp.float32) / HALF))
    angles = jnp.arange(SEQ, dtype=jnp.float32)[:, None] * freqs[None, :]
    cos_il2 = jnp.tile(jnp.repeat(jnp.cos(angles), 2, axis=1), (1, 2))
    sn = jnp.sin(angles)
    sin_sg2 = jnp.tile(jnp.stack([-sn, sn], axis=2).reshape(SEQ, DH), (1, 2))

    qkv = _qkv_call(x2d, norm1_w, qkv_base_p, qspl_t)
    ctx = _attn_call(qkv, cos_il2, sin_sg2)

    x2, h2, sel, wgt = _proj_gate_call(ctx, x2d, out_W, out_b, norm2_w,
                                       gate_W)

    ee = jnp.arange(NUM_EXPERTS)[None, :]
    oh0 = (sel[:, 0:1] == ee).astype(jnp.float32)
    oh1 = (sel[:, 1:2] == ee).astype(jnp.float32)
    ohs = oh0 + oh1
    cums = jnp.cumsum(ohs, axis=0)
    counts = cums[-1]
    excl = cums - ohs
    padded = ((counts + BM - 1) // BM) * BM
    base = jnp.cumsum(padded) - padded
    p0 = jnp.sum(oh0 * (base[None, :] + excl), axis=1).astype(jnp.int32)
    p1 = jnp.sum(oh1 * (base[None, :] + excl + oh0), axis=1).astype(jnp.int32)
    blk_ends = jnp.cumsum(padded) / BM
    eids = jnp.minimum(
        jnp.sum(jnp.arange(NBLK_E)[:, None] >= blk_ends[None, :], axis=1),
        NUM_EXPERTS - 1).astype(jnp.int32)
    w1b = jnp.broadcast_to(wgt[:, 0:1], (SEQ, 128))
    w2b = jnp.broadcast_to(wgt[:, 1:2], (SEQ, 128))

    xg, w_bcast = _sc_dispatch(h2, p0.reshape(32, _DTOK),
                               p1.reshape(32, _DTOK), w1b, w2b)
    t1 = _expert_call(_e1a_body, (xg, e_l1_base, spl1_t),
                      [_row_spec(), _wb_spec(DFF, HID), _ws_spec(DFF, HID)],
                      DFF, eids)
    u = _expert_call(_e1b_body, (xg, t1, e_l2_base, spl2_t),
                     [_row_spec(),
                      pl.BlockSpec((BM, DFF), lambda b, eids: (b, 0)),
                      _wb_spec(DFF, HID), _ws_spec(DFF, HID)], DFF, eids)
    ffw = _expert_call(_e2_body, (u, w_bcast, e_l3_base, spl3_t),
                       [pl.BlockSpec((BM, DFF), lambda b, eids: (b, 0)),
                        pl.BlockSpec((BM, 128), lambda b, eids: (b, 0)),
                        _wb_spec(HID, DFF), _ws_spec(HID, DFF)],
                       HID, eids)

    out2d = _sc_combine(x2, ffw, p0, p1)
    return out2d.reshape(1, SEQ, HID)


# speedup vs baseline: 1.4040x; 1.0628x over previous
import functools

import jax
import jax.numpy as jnp
from jax import lax
from jax.experimental import pallas as pl
from jax.experimental.pallas import tpu as pltpu
from jax.experimental.pallas import tpu_sc as plsc

H_HEADS = 12
GRID_SIZE = 3
SPLINE_ORDER = 3
NUM_EXPERTS = 8
TOP_K = 2
HID = 768
DFF = 768
SEQ = 2048
COEFF = GRID_SIZE + SPLINE_ORDER
NACT = COEFF + 1
KP = NACT * HID
DH = HID // H_HEADS
HALF = DH // 2

BM = 256
NBLK_M = SEQ // BM
BMQ = 512
NBLK_Q = SEQ // BMQ
NPAIR = TOP_K * SEQ
NPAD = NPAIR + NUM_EXPERTS * BM
NBLK_E = NPAD // BM

_H = 2.0 / GRID_SIZE


def _silu(x):
    return x / (1.0 + jnp.exp(-x))


def _dotT(a, w):
    return lax.dot_general(a, w, (((1,), (1,)), ((), ())),
                           preferred_element_type=jnp.float32)


def _dotT16(a, w):
    return lax.dot_general(a.astype(jnp.bfloat16), w.astype(jnp.bfloat16),
                           (((1,), (1,)), ((), ())),
                           preferred_element_type=jnp.float32)


def _spline_bases(x):
    xi = (x + 3.0) * (1.0 / _H)
    j = jnp.floor(xi)
    u = (xi - j).astype(jnp.bfloat16)
    u2 = u * u
    u3 = u2 * u
    v = 1.0 - u
    m0 = u3 * (1.0 / 6.0)
    m1 = (((-3.0 * u + 3.0) * u + 3.0) * u + 1.0) * (1.0 / 6.0)
    m2 = ((3.0 * u - 6.0) * u2 + 4.0) * (1.0 / 6.0)
    m3 = (v * v * v) * (1.0 / 6.0)
    e = [jnp.where(j == k, 1.0, 0.0).astype(jnp.bfloat16)
         for k in range(2 * SPLINE_ORDER + GRID_SIZE)]
    return [e[c] * m0 + e[c + 1] * m1 + e[c + 2] * m2 + e[c + 3] * m3
            for c in range(COEFF)]


def _acts(x):
    return [_silu(x.astype(jnp.bfloat16))] + _spline_bases(x)


def _rmsnorm(x, w, eps=1e-6):
    return w * (x * lax.rsqrt(jnp.mean(x * x, axis=-1, keepdims=True) + eps))


def _qkv_body(x_ref, n1_ref, wb_ref, ws_ref, o_ref, acts_ref, acc_ref):
    c = pl.program_id(1)

    @pl.when(c == 0)
    def _():
        h = _rmsnorm(x_ref[...], n1_ref[...])
        a = _acts(h)
        for i in range(NACT):
            acts_ref[i] = a[i]
        acc_ref[...] = _dotT16(acts_ref[0], wb_ref[...])

    @pl.when(c > 0)
    def _():
        acc_ref[...] += _dotT16(acts_ref[c], ws_ref[0])

    @pl.when(c == NACT - 1)
    def _():
        o_ref[...] = acc_ref[...]


def _qkv_call(x2d, norm1_w, qkv_base, qspl_t):
    return pl.pallas_call(
        _qkv_body,
        grid=(NBLK_Q, NACT),
        in_specs=[
            pl.BlockSpec((BMQ, HID), lambda m, c: (m, 0)),
            pl.BlockSpec((1, HID), lambda m, c: (0, 0)),
            pl.BlockSpec((3 * HID, HID), lambda m, c: (0, 0)),
            pl.BlockSpec((1, 3 * HID, HID),
                         lambda m, c: (jnp.maximum(c - 1, 0), 0, 0)),
        ],
        out_specs=pl.BlockSpec((BMQ, 3 * HID), lambda m, c: (m, 0)),
        out_shape=jax.ShapeDtypeStruct((SEQ, 3 * HID), jnp.float32),
        scratch_shapes=[
            pltpu.VMEM((NACT, BMQ, HID), jnp.bfloat16),
            pltpu.VMEM((BMQ, 3 * HID), jnp.float32),
        ],
    )(x2d, norm1_w.reshape(1, HID), qkv_base, qspl_t)


def _attn_body(q_ref, k_ref, v_ref, cos_ref, sin_ref, o_ref):
    m = pl.program_id(1)
    q2 = q_ref[...]
    k2 = k_ref[...]
    v2 = v_ref[...]
    row = lax.broadcasted_iota(jnp.int32, (2 * DH, 2 * DH), 0)
    col = lax.broadcasted_iota(jnp.int32, (2 * DH, 2 * DH), 1)
    P = jnp.where(col == row - 2 * (row % 2) + 1, 1.0, 0.0).astype(jnp.float32)

    def rot(t, cs, sn):
        return t * cs + jnp.dot(t, P, preferred_element_type=jnp.float32) * sn

    qr = rot(q2, cos_ref[pl.ds(m * BM, BM), :], sin_ref[pl.ds(m * BM, BM), :])
    kr = rot(k2, cos_ref[...], sin_ref[...])
    ctxs = []
    for s in range(2):
        sl = slice(s * DH, (s + 1) * DH)
        scores = lax.dot_general(qr[:, sl], kr[:, sl], (((1,), (1,)), ((), ())),
                                 preferred_element_type=jnp.float32)
        scores = scores * (1.0 / (DH ** 0.5))
        mx = jnp.max(scores, axis=1, keepdims=True)
        p = jnp.exp(scores - mx)
        attn = p * (1.0 / jnp.sum(p, axis=1, keepdims=True))
        ctxs.append(jnp.dot(attn, v2[:, sl], preferred_element_type=jnp.float32))
    o_ref[...] = jnp.concatenate(ctxs, axis=1)


def _attn_call(qkv, cos_il2, sin_sg2):
    hp = H_HEADS // 2
    return pl.pallas_call(
        _attn_body,
        grid=(hp, NBLK_M),
        in_specs=[
            pl.BlockSpec((BM, 2 * DH), lambda h, m: (m, h)),
            pl.BlockSpec((SEQ, 2 * DH), lambda h, m: (0, hp + h)),
            pl.BlockSpec((SEQ, 2 * DH), lambda h, m: (0, 2 * hp + h)),
            pl.BlockSpec((SEQ, 2 * DH), lambda h, m: (0, 0)),
            pl.BlockSpec((SEQ, 2 * DH), lambda h, m: (0, 0)),
        ],
        out_specs=pl.BlockSpec((BM, 2 * DH), lambda h, m: (m, h)),
        out_shape=jax.ShapeDtypeStruct((SEQ, HID), jnp.float32),
    )(qkv, qkv, qkv, cos_il2, sin_sg2)


def _proj_gate_body(ctx_ref, x_ref, w_ref, b_ref, n2_ref, gw_ref,
                    x2_ref, h2_ref, sel_ref, wgt_ref):
    o = _dotT(ctx_ref[...], w_ref[...])
    x2 = x_ref[...] + o + b_ref[...]
    x2_ref[...] = x2
    h2 = _rmsnorm(x2, n2_ref[...])
    h2_ref[...] = h2
    logits = _dotT(h2, gw_ref[...])
    iota = lax.broadcasted_iota(jnp.int32, logits.shape, 1)
    m1 = jnp.max(logits, axis=1, keepdims=True)
    e1 = jnp.min(jnp.where(logits == m1, iota, NUM_EXPERTS),
                 axis=1, keepdims=True)
    masked = jnp.where(iota == e1, -jnp.inf, logits)
    m2 = jnp.max(masked, axis=1, keepdims=True)
    e2 = jnp.min(jnp.where(masked == m2, iota, NUM_EXPERTS),
                 axis=1, keepdims=True)
    t = jnp.exp(m2 - m1)
    w1 = 1.0 / (1.0 + t)
    w2 = 1.0 - w1
    sel_ref[...] = jnp.where(iota == 0, e1, jnp.where(iota == 1, e2, 0))
    wgt_ref[...] = jnp.where(iota == 0, w1, jnp.where(iota == 1, w2, 0.0))


def _proj_gate_call(ctx, x2d, wproj, out_b, norm2_w, gate_W):
    return pl.pallas_call(
        _proj_gate_body,
        grid=(NBLK_M,),
        in_specs=[
            pl.BlockSpec((BM, HID), lambda m: (m, 0)),
            pl.BlockSpec((BM, HID), lambda m: (m, 0)),
            pl.BlockSpec((HID, HID), lambda m: (0, 0)),
            pl.BlockSpec((1, HID), lambda m: (0, 0)),
            pl.BlockSpec((1, HID), lambda m: (0, 0)),
            pl.BlockSpec((NUM_EXPERTS, HID), lambda m: (0, 0)),
        ],
        out_specs=[
            pl.BlockSpec((BM, HID), lambda m: (m, 0)),
            pl.BlockSpec((BM, HID), lambda m: (m, 0)),
            pl.BlockSpec((BM, NUM_EXPERTS), lambda m: (m, 0)),
            pl.BlockSpec((BM, NUM_EXPERTS), lambda m: (m, 0)),
        ],
        out_shape=[
            jax.ShapeDtypeStruct((SEQ, HID), jnp.float32),
            jax.ShapeDtypeStruct((SEQ, HID), jnp.float32),
            jax.ShapeDtypeStruct((SEQ, NUM_EXPERTS), jnp.int32),
            jax.ShapeDtypeStruct((SEQ, NUM_EXPERTS), jnp.float32),
        ],
    )(ctx, x2d, wproj, out_b.reshape(1, HID), norm2_w.reshape(1, HID),
      gate_W)


def _sc_mesh():
    return plsc.VectorSubcoreMesh(core_axis_name="c", subcore_axis_name="s")


_DTOK = SEQ // 32


def _sc_dispatch(h2, p0, p1, w1b, w2b):
    @functools.partial(
        pl.kernel, mesh=_sc_mesh(),
        out_type=(jax.ShapeDtypeStruct((NPAD, HID), jnp.float32),
                  jax.ShapeDtypeStruct((NPAD, 128), jnp.float32)),
        scratch_types=[
            pltpu.VMEM((_DTOK, HID), jnp.float32),
            pltpu.VMEM((_DTOK, 128), jnp.float32),
            pltpu.VMEM((1, _DTOK), jnp.int32),
            pltpu.VMEM((1, _DTOK), jnp.int32),
        ],
    )
    def k(h2_hbm, p0_hbm, p1_hbm, w1_hbm, w2_hbm, xg_hbm, wp_hbm,
          rows_v, wrow_v, i0, i1):
        wid = lax.axis_index("s") * 2 + lax.axis_index("c")
        pltpu.sync_copy(p0_hbm.at[pl.ds(wid, 1)], i0)
        pltpu.sync_copy(p1_hbm.at[pl.ds(wid, 1)], i1)
        pltpu.sync_copy(h2_hbm.at[pl.ds(wid * _DTOK, _DTOK)], rows_v)
        pltpu.sync_copy(rows_v, xg_hbm.at[i0.at[0]])
        pltpu.sync_copy(rows_v, xg_hbm.at[i1.at[0]])
        pltpu.sync_copy(w1_hbm.at[pl.ds(wid * _DTOK, _DTOK)], wrow_v)
        pltpu.sync_copy(wrow_v, wp_hbm.at[i0.at[0]])
        pltpu.sync_copy(w2_hbm.at[pl.ds(wid * _DTOK, _DTOK)], wrow_v)
        pltpu.sync_copy(wrow_v, wp_hbm.at[i1.at[0]])

    return k(h2, p0, p1, w1b, w2b)


_CTOK = 32
_TPT = SEQ // 32
_NCC = _TPT // _CTOK


def _sc_combine(x2, ffw, p0, p1):
    @functools.partial(
        pl.kernel, mesh=_sc_mesh(),
        out_type=jax.ShapeDtypeStruct((SEQ, HID), jnp.float32),
        scratch_types=[
            pltpu.VMEM((_CTOK, HID), jnp.float32),
            pltpu.VMEM((_CTOK, HID), jnp.float32),
            pltpu.VMEM((_CTOK, HID), jnp.float32),
            pltpu.VMEM((_CTOK,), jnp.int32),
            pltpu.VMEM((_CTOK,), jnp.int32),
            pltpu.SemaphoreType.DMA,
        ],
    )
    def k(x2_hbm, ffw_hbm, p0_hbm, p1_hbm, out_hbm, acc, r0, r1, i0, i1, sem):
        wid = lax.axis_index("s") * 2 + lax.axis_index("c")
        for j in range(_NCC):
            base = wid * _TPT + j * _CTOK
            pltpu.sync_copy(p0_hbm.at[pl.ds(base, _CTOK)], i0)
            pltpu.sync_copy(p1_hbm.at[pl.ds(base, _CTOK)], i1)
            pltpu.sync_copy(x2_hbm.at[pl.ds(base, _CTOK)], acc)
            pltpu.async_copy(ffw_hbm.at[i0], r0, sem).wait()
            pltpu.async_copy(ffw_hbm.at[i1], r1, sem).wait()

            @pl.loop(0, _CTOK)
            def _(t):
                for c in range(HID // 16):
                    sl = pl.ds(c * 16, 16)
                    acc[t, sl] += r0[t, sl] + r1[t, sl]

            pltpu.sync_copy(acc, out_hbm.at[pl.ds(base, _CTOK)])

    return k(x2, ffw, p0, p1)


def _kan_acc(x, wb_ref, ws_ref):
    a = _acts(x)
    acc = _dotT16(a[0], wb_ref[0])
    for c in range(COEFF):
        acc += _dotT16(a[c + 1], ws_ref[0, c])
    return acc


def _e1a_body(eids_ref, xg_ref, wb_ref, ws_ref, o_ref):
    del eids_ref
    o_ref[...] = _kan_acc(xg_ref[...], wb_ref, ws_ref)


def _e1b_body(eids_ref, xg_ref, t1_ref, wb_ref, ws_ref, o_ref):
    del eids_ref
    o_ref[...] = _kan_acc(xg_ref[...], wb_ref, ws_ref) * t1_ref[...]


def _e2_body(eids_ref, u_ref, wbc_ref, wb_ref, ws_ref, o_ref):
    del eids_ref
    o_ref[...] = _kan_acc(u_ref[...], wb_ref, ws_ref) * wbc_ref[:, 0:1]


def _row_spec():
    return pl.BlockSpec((BM, HID), lambda b, eids: (b, 0))


def _wb_spec(out_d, in_d):
    return pl.BlockSpec((1, out_d, in_d), lambda b, eids: (eids[b], 0, 0))


def _ws_spec(out_d, in_d):
    return pl.BlockSpec((1, COEFF, out_d, in_d),
                        lambda b, eids: (eids[b], 0, 0, 0))


def _expert_call(body, ins, in_specs, out_dim, eids):
    grid_spec = pltpu.PrefetchScalarGridSpec(
        num_scalar_prefetch=1,
        grid=(NBLK_E,),
        in_specs=in_specs,
        out_specs=pl.BlockSpec((BM, out_dim), lambda b, eids: (b, 0)),
    )
    return pl.pallas_call(
        body,
        grid_spec=grid_spec,
        out_shape=jax.ShapeDtypeStruct((NPAD, out_dim), jnp.float32),
    )(eids, *ins)


def kernel(x, norm1_w, norm2_w, qkv_base, qkv_spline, out_W, out_b, gate_W,
           e_l1_base, e_l1_spline, e_l2_base, e_l2_spline, e_l3_base,
           e_l3_spline):
    x2d = x.reshape(SEQ, HID)

    r = jnp.arange(3 * HID)
    rowperm = (r % 768 // DH) * (3 * DH) + (r // 768) * DH + r % DH
    qkv_base_p = qkv_base[rowperm]
    qspl_t = qkv_spline.transpose(2, 0, 1)[:, rowperm, :]
    spl1_t = e_l1_spline.transpose(0, 3, 1, 2)
    spl2_t = e_l2_spline.transpose(0, 3, 1, 2)
    spl3_t = e_l3_spline.transpose(0, 3, 1, 2)
    freqs = 1.0 / (10000.0 ** (jnp.arange(HALF, dtype=jnp.float32) / HALF))
    angles = jnp.arange(SEQ, dtype=jnp.float32)[:, None] * freqs[None, :]
    cos_il2 = jnp.tile(jnp.repeat(jnp.cos(angles), 2, axis=1), (1, 2))
    sn = jnp.sin(angles)
    sin_sg2 = jnp.tile(jnp.stack([-sn, sn], axis=2).reshape(SEQ, DH), (1, 2))

    qkv = _qkv_call(x2d, norm1_w, qkv_base_p, qspl_t)
    ctx = _attn_call(qkv, cos_il2, sin_sg2)

    x2, h2, sel, wgt = _proj_gate_call(ctx, x2d, out_W, out_b, norm2_w,
                                       gate_W)

    ee = jnp.arange(NUM_EXPERTS)[None, :]
    oh0 = (sel[:, 0:1] == ee).astype(jnp.float32)
    oh1 = (sel[:, 1:2] == ee).astype(jnp.float32)
    ohs = oh0 + oh1
    cums = jnp.cumsum(ohs, axis=0)
    counts = cums[-1]
    excl = cums - ohs
    padded = ((counts + BM - 1) // BM) * BM
    base = jnp.cumsum(padded) - padded
    p0 = jnp.sum(oh0 * (base[None, :] + excl), axis=1).astype(jnp.int32)
    p1 = jnp.sum(oh1 * (base[None, :] + excl + oh0), axis=1).astype(jnp.int32)
    blk_ends = jnp.cumsum(padded) / BM
    eids = jnp.minimum(
        jnp.sum(jnp.arange(NBLK_E)[:, None] >= blk_ends[None, :], axis=1),
        NUM_EXPERTS - 1).astype(jnp.int32)
    w1b = jnp.broadcast_to(wgt[:, 0:1], (SEQ, 128))
    w2b = jnp.broadcast_to(wgt[:, 1:2], (SEQ, 128))

    xg, w_bcast = _sc_dispatch(h2, p0.reshape(32, _DTOK),
                               p1.reshape(32, _DTOK), w1b, w2b)
    t1 = _expert_call(_e1a_body, (xg, e_l1_base, spl1_t),
                      [_row_spec(), _wb_spec(DFF, HID), _ws_spec(DFF, HID)],
                      DFF, eids)
    u = _expert_call(_e1b_body, (xg, t1, e_l2_base, spl2_t),
                     [_row_spec(),
                      pl.BlockSpec((BM, DFF), lambda b, eids: (b, 0)),
                      _wb_spec(DFF, HID), _ws_spec(DFF, HID)], DFF, eids)
    ffw = _expert_call(_e2_body, (u, w_bcast, e_l3_base, spl3_t),
                       [pl.BlockSpec((BM, DFF), lambda b, eids: (b, 0)),
                        pl.BlockSpec((BM, 128), lambda b, eids: (b, 0)),
                        _wb_spec(HID, DFF), _ws_spec(HID, DFF)],
                       HID, eids)

    out2d = _sc_combine(x2, ffw, p0, p1)
    return out2d.reshape(1, SEQ, HID)
